# baseline (speedup 1.0000x reference)
import functools

import jax
import jax.numpy as jnp
from jax import lax
from jax.experimental import pallas as pl
from jax.experimental.pallas import tpu as pltpu
from jax.experimental.pallas import tpu_sc as plsc

BATCH = 16384
DIM = 64
NC, NS = 2, 16
NW = NC * NS
B_PER_W = BATCH // NW


PROWS = 500000


def _sc_gather(uidx, iidx, uq, iq, gu, gi, mu_p, mi_p):
    mesh = plsc.VectorSubcoreMesh(core_axis_name="c", subcore_axis_name="s")
    HALF = B_PER_W // 4

    @functools.partial(
        pl.kernel,
        out_type=[jax.ShapeDtypeStruct((BATCH, DIM), jnp.float32),
                  jax.ShapeDtypeStruct((BATCH, DIM), jnp.float32),
                  jax.ShapeDtypeStruct((BATCH, 128), jnp.float32),
                  jax.ShapeDtypeStruct((BATCH, 128), jnp.float32)],
        mesh=mesh,
        scratch_types=[
            pltpu.VMEM((B_PER_W,), jnp.int32),
            pltpu.VMEM((B_PER_W,), jnp.int32),
            pltpu.VMEM((B_PER_W,), jnp.int32),
            pltpu.VMEM((B_PER_W,), jnp.int32),
            pltpu.VMEM((HALF, DIM), jnp.float32),
            pltpu.VMEM((HALF, DIM), jnp.float32),
            pltpu.VMEM((HALF, 128), jnp.float32),
            pltpu.VMEM((HALF, 128), jnp.float32),
            pltpu.SemaphoreType.DMA,
            pltpu.SemaphoreType.DMA,
        ],
    )
    def k(uidx_hbm, iidx_hbm, uq_hbm, iq_hbm, gu_hbm, gi_hbm, mu_hbm, mi_hbm,
          gu_out, gi_out, mu_out, mi_out,
          uidx_v, iidx_v, uq_v, iq_v, buf_a, buf_b, buf_c, buf_d,
          sem_a, sem_b):
        wid = lax.axis_index("s") * NC + lax.axis_index("c")
        base = wid * B_PER_W
        pltpu.sync_copy(uidx_hbm.at[pl.ds(base, B_PER_W)], uidx_v)
        pltpu.sync_copy(iidx_hbm.at[pl.ds(base, B_PER_W)], iidx_v)
        pltpu.sync_copy(uq_hbm.at[pl.ds(base, B_PER_W)], uq_v)
        pltpu.sync_copy(iq_hbm.at[pl.ds(base, B_PER_W)], iq_v)

        jobs = []
        for table, idx_v, out, bufs in ((gu_hbm, uidx_v, gu_out,
                                         (buf_a, buf_b)),
                                        (gi_hbm, iidx_v, gi_out,
                                         (buf_a, buf_b)),
                                        (mu_hbm, uq_v, mu_out,
                                         (buf_c, buf_d)),
                                        (mi_hbm, iq_v, mi_out,
                                         (buf_c, buf_d))):
            for h in range(B_PER_W // HALF):
                jobs.append((table, idx_v, out, bufs, h))

        sems = (sem_a, sem_b)

        def fire(table, idx_v, buf, sem, h):
            def body(g, _):
                vec = idx_v[pl.ds(h * HALF + g * 16, 16)]
                for lane in range(16):
                    i = vec[lane]
                    pltpu.async_copy(table.at[pl.ds(i, 1)],
                                     buf.at[pl.ds(g * 16 + lane, 1)], sem)
                return _
            lax.fori_loop(0, HALF // 16, body, 0)

        def drain_and_write(n):
            table, idx_v, out, bufs, h = jobs[n]
            s = n % 2
            pltpu.make_async_copy(out.at[pl.ds(0, HALF)], bufs[s],
                                  sems[s]).wait()
            pltpu.sync_copy(bufs[s],
                            out.at[pl.ds(base + h * HALF, HALF)])

        for n, (table, idx_v, out, bufs, h) in enumerate(jobs):
            if n >= 2:
                drain_and_write(n - 2)
            fire(table, idx_v, bufs[n % 2], sems[n % 2], h)
        drain_and_write(len(jobs) - 2)
        drain_and_write(len(jobs) - 1)

    return k(uidx, iidx, uq, iq, gu, gi, mu_p, mi_p)


BM = 2048


def _sel_half(slab_ref, sel2):
    x = slab_ref[...]
    m0 = (sel2[:, 0:1] != 0).astype(jnp.float32)
    m1 = (sel2[:, 1:2] != 0).astype(jnp.float32)
    return x[:, :DIM] * m0 + x[:, DIM:] * m1


def _tc_mlp(gu_rows, gi_rows, mu_slabs, mi_slabs, selu2, seli2,
            W1, b1, W2, b2, Wf, bf):
    def body(gu_ref, gi_ref, mu_ref, mi_ref, selu_ref, seli_ref,
             w1_ref, b1_ref, w2_ref, b2_ref, wf_ref, bf_ref, out_ref):
        gmf = gu_ref[...] * gi_ref[...]
        mu = _sel_half(mu_ref, selu_ref[...])
        mi = _sel_half(mi_ref, seli_ref[...])
        w1 = w1_ref[...]
        h = jnp.dot(mu, w1[:DIM], preferred_element_type=jnp.float32)
        h = h + jnp.dot(mi, w1[DIM:],
                        preferred_element_type=jnp.float32)
        h = jnp.maximum(h + b1_ref[...], 0.0)
        h = jnp.maximum(
            jnp.dot(h, w2_ref[...], preferred_element_type=jnp.float32)
            + b2_ref[...], 0.0)
        wf = wf_ref[...]
        pred = (jnp.dot(gmf, wf[:DIM], preferred_element_type=jnp.float32)
                + jnp.dot(h, wf[DIM:], preferred_element_type=jnp.float32)
                + bf_ref[...])
        out_ref[...] = pred

    grid = (BATCH // BM,)
    rows_spec = pl.BlockSpec((BM, DIM), lambda i: (i, 0))
    slab_spec = pl.BlockSpec((BM, 128), lambda i: (i, 0))
    sel_spec = pl.BlockSpec((BM, 2), lambda i: (i, 0))
    full = lambda shape: pl.BlockSpec(shape, lambda i: (0,) * len(shape))
    return pl.pallas_call(
        body,
        grid=grid,
        in_specs=[
            rows_spec, rows_spec, slab_spec, slab_spec,
            sel_spec, sel_spec,
            full((2 * DIM, DIM)), full((1, DIM)),
            full((DIM, 32)), full((1, 32)),
            full((DIM + 32, 1)), full((1, 1)),
        ],
        out_specs=pl.BlockSpec((BM, 1), lambda i: (i, 0)),
        out_shape=jax.ShapeDtypeStruct((BATCH, 1), jnp.float32),
    )(gu_rows, gi_rows, mu_slabs, mi_slabs, selu2, seli2,
      W1, b1, W2, b2, Wf, bf)


def _onehot(v, n):
    return (jnp.arange(n, dtype=jnp.int32)[None, :]
            == v[:, None]).astype(jnp.int32)


def kernel(user_ids, item_ids, gmf_user_w, gmf_item_w, mlp_user_w, mlp_item_w,
           W1, b1, W2, b2, Wf, bf):
    uidx = user_ids.astype(jnp.int32)
    iidx = item_ids.astype(jnp.int32)
    gu, gi, mu_s, mi_s = _sc_gather(
        uidx, iidx, uidx // 2, iidx // 2,
        gmf_user_w, gmf_item_w,
        mlp_user_w.reshape(PROWS, 128), mlp_item_w.reshape(PROWS, 128))
    pred = _tc_mlp(gu, gi, mu_s, mi_s,
                   _onehot(uidx % 2, 2), _onehot(iidx % 2, 2),
                   W1, b1.reshape(1, DIM), W2, b2.reshape(1, 32),
                   Wf, bf.reshape(1, 1))
    return pred[:, 0]
